# Initial kernel scaffold; baseline (speedup 1.0000x reference)
import jax
import jax.numpy as jnp
from jax import lax
from jax.experimental import pallas as pl
from jax.experimental.pallas import tpu as pltpu
from jax.experimental.pallas import tpu_sc as plsc

N = 10000
E = 160000
D = 256
DC = 128
NTILES = 16
ET = E // NTILES
B = 125
NB = ET // B
RT = N // NTILES


def _sc_segment_kernel(x0, x1, h0, h1, src3, dst3, w3, z128, z8, ones8):
    mesh = plsc.VectorSubcoreMesh(core_axis_name="c", subcore_axis_name="s")

    def body(x0_hbm, x1_hbm, h0_hbm, h1_hbm, src_hbm, dst_hbm, w_hbm,
             z128_hbm, z8_hbm, ones8_hbm, agg_out, deg_out,
             src_v, dst_v, w_v, msg_v, ones8_v, acc_s, deg_s):
        cid = lax.axis_index("c")
        tid = lax.axis_index("s")
        row0 = tid * RT

        pltpu.sync_copy(src_hbm.at[tid], src_v)
        pltpu.sync_copy(dst_hbm.at[tid], dst_v)
        pltpu.sync_copy(w_hbm.at[tid], w_v)
        pltpu.sync_copy(ones8_hbm, ones8_v)

        def do_chunk(chunk_hbm, chunk_idx, with_deg):
            pltpu.sync_copy(z128_hbm, acc_s.at[pl.ds(row0, RT)])
            if with_deg:
                pltpu.sync_copy(z8_hbm, deg_s.at[pl.ds(row0, RT)])
            plsc.subcore_barrier()

            def batch_body(b):
                pltpu.sync_copy(chunk_hbm.at[src_v.at[b]], msg_v)

                def edge_body(e, _):
                    we = w_v[b, e]
                    for j in range(DC // 16):
                        sl = pl.ds(j * 16, 16)
                        msg_v[e, sl] = msg_v[e, sl] * we
                    return 0

                lax.fori_loop(0, B, edge_body, 0)

                pltpu.sync_copy(msg_v, acc_s.at[dst_v.at[b]], add=True)
                if with_deg:
                    pltpu.sync_copy(ones8_v, deg_s.at[dst_v.at[b]], add=True)

            lax.fori_loop(0, NB, lambda b, _: (batch_body(b), 0)[1], 0)
            plsc.subcore_barrier()

            pltpu.sync_copy(acc_s.at[pl.ds(row0, RT)],
                            agg_out.at[chunk_idx, pl.ds(row0, RT)])
            if with_deg:
                pltpu.sync_copy(deg_s.at[pl.ds(row0, RT)],
                                deg_out.at[pl.ds(row0, RT)])

        @pl.when(cid == 0)
        def _():
            do_chunk(x0_hbm, 0, True)
            do_chunk(x1_hbm, 1, False)

        @pl.when(cid == 1)
        def _():
            do_chunk(h0_hbm, 2, False)
            do_chunk(h1_hbm, 3, False)

    f = pl.kernel(
        body,
        out_type=(jax.ShapeDtypeStruct((4, N, DC), jnp.float32),
                  jax.ShapeDtypeStruct((N, 8), jnp.float32)),
        mesh=mesh,
        scratch_types=[
            pltpu.VMEM((NB, B), jnp.int32),
            pltpu.VMEM((NB, B), jnp.int32),
            pltpu.VMEM((NB, B), jnp.float32),
            pltpu.VMEM((B, DC), jnp.float32),
            pltpu.VMEM((B, 8), jnp.float32),
            pltpu.VMEM_SHARED((N, DC), jnp.float32),
            pltpu.VMEM_SHARED((N, 8), jnp.float32),
        ],
    )
    return f(x0, x1, h0, h1, src3, dst3, w3, z128, z8, ones8)


def _tc_body(a0, a1, a2, a3, deg8, x, h, win, wis, whn, whs, bi, bh, out):
    deg = jnp.maximum(deg8[:, 0:1], 1.0)
    m0 = a0[...] / deg
    m1 = a1[...] / deg
    m2 = a2[...] / deg
    m3 = a3[...] / deg
    hp = lax.Precision.HIGHEST
    acc = jnp.dot(x[...], wis[...], precision=hp)
    acc += jnp.dot(h[...], whs[...], precision=hp)
    acc += jnp.dot(m0, win[0:DC, :], precision=hp)
    acc += jnp.dot(m1, win[DC:D, :], precision=hp)
    acc += jnp.dot(m2, whn[0:DC, :], precision=hp)
    acc += jnp.dot(m3, whn[DC:D, :], precision=hp)
    out[...] = jnp.tanh(acc + bi[...] + bh[...])


def _tc_dense(a0, a1, a2, a3, deg8, X, H, Wi_neigh, Wi_self, Wh_neigh,
              Wh_self, bi, bh):
    R = 400
    grid = (N // R,)

    def row_spec(w):
        return pl.BlockSpec((R, w), lambda i: (i, 0))

    def full_spec(r, c):
        return pl.BlockSpec((r, c), lambda i: (0, 0))

    return pl.pallas_call(
        _tc_body,
        grid=grid,
        in_specs=[
            row_spec(DC), row_spec(DC), row_spec(DC), row_spec(DC),
            row_spec(8), row_spec(D), row_spec(D),
            full_spec(D, D), full_spec(D, D), full_spec(D, D),
            full_spec(D, D), full_spec(1, D), full_spec(1, D),
        ],
        out_specs=row_spec(D),
        out_shape=jax.ShapeDtypeStruct((N, D), jnp.float32),
    )(a0, a1, a2, a3, deg8, X, H, Wi_neigh, Wi_self, Wh_neigh, Wh_self,
      bi.reshape(1, D), bh.reshape(1, D))


def kernel(X, edge_index, edge_weight, H, Wi_neigh, Wi_self, bi, Wh_neigh,
           Wh_self, bh):
    src = edge_index[0].reshape(NTILES, NB, B)
    dst = edge_index[1].reshape(NTILES, NB, B)
    w = edge_weight.reshape(NTILES, NB, B)
    x0 = X[:, :DC]
    x1 = X[:, DC:]
    h0 = H[:, :DC]
    h1 = H[:, DC:]
    z128 = jnp.zeros((RT, DC), jnp.float32)
    z8 = jnp.zeros((RT, 8), jnp.float32)
    ones8 = jnp.ones((B, 8), jnp.float32)

    agg, deg8 = _sc_segment_kernel(x0, x1, h0, h1, src, dst, w, z128, z8,
                                   ones8)
    return _tc_dense(agg[0], agg[1], agg[2], agg[3], deg8, X, H, Wi_neigh,
                     Wi_self, Wh_neigh, Wh_self, bi, bh)


# speedup vs baseline: 2.9778x; 2.9778x over previous
import jax
import jax.numpy as jnp
from jax import lax
from jax.experimental import pallas as pl
from jax.experimental.pallas import tpu as pltpu
from jax.experimental.pallas import tpu_sc as plsc

N = 10000
E = 160000
D = 256
DC = 128
NTILES = 16
ET = E // NTILES
B = 128
ETP = 10240
NB = ETP // B
SBN = 8
NSB = NB // SBN
NP = 10240
RT = NP // NTILES


def _sc_segment_kernel(x0, x1, h0, h1, src3, dst3, w3, ones1):
    mesh = plsc.VectorSubcoreMesh(core_axis_name="c", subcore_axis_name="s")

    def body(x0_hbm, x1_hbm, h0_hbm, h1_hbm, src_hbm, dst_hbm, w_hbm,
             ones_hbm, agg_out, deg_out, src_v, dst_v, w_v, msg_v, ones_v,
             acc_s, deg_s):
        cid = lax.axis_index("c")
        tid = lax.axis_index("s")
        row0 = tid * RT

        def zero_msg(r, _):
            for q in range(DC // 16):
                msg_v[r, pl.ds(q * 16, 16)] = jnp.zeros((16,), jnp.float32)
            return 0

        pltpu.sync_copy(ones_hbm, ones_v)
        rowd = tid * (NP // NTILES)

        def do_chunk(chunk_hbm, chunk_idx, with_deg):
            lax.fori_loop(0, B, zero_msg, 0)
            for i in range(RT // B):
                pltpu.sync_copy(msg_v, acc_s.at[pl.ds(row0 + i * B, B)])
                if with_deg:
                    pltpu.sync_copy(msg_v.at[0],
                                    deg_s.at[pl.ds(rowd + i * B, B)])
            plsc.subcore_barrier()

            def super_body(sb, _):
                pltpu.sync_copy(src_hbm.at[tid, pl.ds(sb * SBN, SBN)], src_v)
                pltpu.sync_copy(dst_hbm.at[tid, pl.ds(sb * SBN, SBN)], dst_v)
                pltpu.sync_copy(w_hbm.at[tid, pl.ds(sb * SBN, SBN)], w_v)

                def batch_body(j, _):
                    pltpu.sync_copy(chunk_hbm.at[src_v.at[j]], msg_v)

                    def group_body(g, _):
                        wv = w_v[j, pl.ds(g * 16, 16)]
                        for k in range(16):
                            we = wv[k]
                            for q in range(DC // 16):
                                sl = pl.ds(q * 16, 16)
                                msg_v[g * 16 + k, sl] = (
                                    msg_v[g * 16 + k, sl] * we)
                        return 0

                    lax.fori_loop(0, B // 16, group_body, 0)

                    pltpu.sync_copy(msg_v, acc_s.at[dst_v.at[j]], add=True)
                    if with_deg:
                        pltpu.sync_copy(ones_v, deg_s.at[dst_v.at[j]],
                                        add=True)
                    return 0

                lax.fori_loop(0, SBN, batch_body, 0)
                return 0

            lax.fori_loop(0, NSB, super_body, 0)
            plsc.subcore_barrier()

            for i in range(RT // B):
                pltpu.sync_copy(acc_s.at[pl.ds(row0 + i * B, B)], msg_v)
                pltpu.sync_copy(msg_v,
                                agg_out.at[chunk_idx,
                                           pl.ds(row0 + i * B, B)])
            if with_deg:
                for i in range(RT // B):
                    pltpu.sync_copy(deg_s.at[pl.ds(rowd + i * B, B)],
                                    ones_v)
                    pltpu.sync_copy(ones_v,
                                    deg_out.at[pl.ds(rowd + i * B, B)])

        @pl.when(cid == 0)
        def _():
            do_chunk(x0_hbm, 0, True)
            do_chunk(x1_hbm, 1, False)

        @pl.when(cid == 1)
        def _():
            do_chunk(h0_hbm, 2, False)
            do_chunk(h1_hbm, 3, False)

    f = pl.kernel(
        body,
        out_type=(jax.ShapeDtypeStruct((4, NP, DC), jnp.float32),
                  jax.ShapeDtypeStruct((NP,), jnp.float32)),
        mesh=mesh,
        scratch_types=[
            pltpu.VMEM((SBN, B), jnp.int32),
            pltpu.VMEM((SBN, B), jnp.int32),
            pltpu.VMEM((SBN, B), jnp.float32),
            pltpu.VMEM((B, DC), jnp.float32),
            pltpu.VMEM((B,), jnp.float32),
            pltpu.VMEM_SHARED((NP, DC), jnp.float32),
            pltpu.VMEM_SHARED((NP,), jnp.float32),
        ],
    )
    return f(x0, x1, h0, h1, src3, dst3, w3, ones1)


def _tc_body(a0, a1, a2, a3, deg1, x, h, win, wis, whn, whs, bi, bh, out):
    deg = jnp.maximum(deg1[...], 1.0)
    m0 = a0[...] / deg
    m1 = a1[...] / deg
    m2 = a2[...] / deg
    m3 = a3[...] / deg
    hp = lax.Precision.HIGHEST
    acc = jnp.dot(x[...], wis[...], precision=hp)
    acc += jnp.dot(h[...], whs[...], precision=hp)
    acc += jnp.dot(m0, win[0:DC, :], precision=hp)
    acc += jnp.dot(m1, win[DC:D, :], precision=hp)
    acc += jnp.dot(m2, whn[0:DC, :], precision=hp)
    acc += jnp.dot(m3, whn[DC:D, :], precision=hp)
    out[...] = jnp.tanh(acc + bi[...] + bh[...])


def _tc_dense(a0, a1, a2, a3, deg1, X, H, Wi_neigh, Wi_self, Wh_neigh,
              Wh_self, bi, bh):
    R = 400
    grid = (N // R,)

    def row_spec(w):
        return pl.BlockSpec((R, w), lambda i: (i, 0))

    def full_spec(r, c):
        return pl.BlockSpec((r, c), lambda i: (0, 0))

    return pl.pallas_call(
        _tc_body,
        grid=grid,
        in_specs=[
            row_spec(DC), row_spec(DC), row_spec(DC), row_spec(DC),
            pl.BlockSpec((R, 1), lambda i: (i, 0)),
            row_spec(D), row_spec(D),
            full_spec(D, D), full_spec(D, D), full_spec(D, D),
            full_spec(D, D), full_spec(1, D), full_spec(1, D),
        ],
        out_specs=row_spec(D),
        out_shape=jax.ShapeDtypeStruct((N, D), jnp.float32),
    )(a0, a1, a2, a3, deg1, X, H, Wi_neigh, Wi_self, Wh_neigh, Wh_self,
      bi.reshape(1, D), bh.reshape(1, D))


def kernel(X, edge_index, edge_weight, H, Wi_neigh, Wi_self, bi, Wh_neigh,
           Wh_self, bh):
    pad = ETP - ET
    src = jnp.pad(edge_index[0].reshape(NTILES, ET), ((0, 0), (0, pad)),
                  constant_values=0).reshape(NTILES, NB, B)
    dst = jnp.pad(edge_index[1].reshape(NTILES, ET), ((0, 0), (0, pad)),
                  constant_values=NP - 1).reshape(NTILES, NB, B)
    w = jnp.pad(edge_weight.reshape(NTILES, ET), ((0, 0), (0, pad)),
                constant_values=0.0).reshape(NTILES, NB, B)
    x0 = X[:, :DC]
    x1 = X[:, DC:]
    h0 = H[:, :DC]
    h1 = H[:, DC:]

    ones1 = jnp.ones((B,), jnp.float32)
    agg, deg = _sc_segment_kernel(x0, x1, h0, h1, src, dst, w, ones1)
    agg = agg[:, :N]
    deg1 = deg[:N].reshape(N, 1)
    return _tc_dense(agg[0], agg[1], agg[2], agg[3], deg1, X, H, Wi_neigh,
                     Wi_self, Wh_neigh, Wh_self, bi, bh)


# speedup vs baseline: 3.5295x; 1.1853x over previous
import jax
import jax.numpy as jnp
from jax import lax
from jax.experimental import pallas as pl
from jax.experimental.pallas import tpu as pltpu
from jax.experimental.pallas import tpu_sc as plsc

N = 10000
E = 160000
D = 256
DC = 128
NTILES = 16
ET = E // NTILES
B = 128
ETP = 10240
NB = ETP // B
SBN = 8
NSB = NB // SBN
NP = 10240
RT = NP // NTILES


def _sc_segment_kernel(xh, src3, dst3, w3, ones1):
    mesh = plsc.VectorSubcoreMesh(core_axis_name="c", subcore_axis_name="s")

    def body(xh_hbm, src_hbm, dst_hbm, w_hbm, ones_hbm, agg_out, deg_out,
             src_v, dst_v, w_v, msg_a, msg_b, ones_v,
             sem_ga, sem_gb, sem_sa, sem_sb, sem_d, acc_s, deg_s):
        cid = lax.axis_index("c")
        tid = lax.axis_index("s")
        row0 = tid * RT
        msgs = (msg_a, msg_b)
        gsems = (sem_ga, sem_gb)
        ssems = (sem_sa, sem_sb)

        pltpu.sync_copy(ones_hbm, ones_v)

        def zero_msg_a(r, _):
            for q in range(DC // 16):
                msg_a[r, pl.ds(q * 16, 16)] = jnp.zeros((16,), jnp.float32)
            return 0

        def scale(buf, j):
            def group_body(g, _):
                wv = w_v[j, pl.ds(g * 16, 16)]
                for k in range(16):
                    we = wv[k]
                    for q in range(DC // 16):
                        sl = pl.ds(q * 16, 16)
                        buf[g * 16 + k, sl] = buf[g * 16 + k, sl] * we
                return 0

            lax.fori_loop(0, B // 16, group_body, 0)

        def do_pass(p, _):
            with_deg = jnp.logical_and(cid == 0, p == 0)
            chunk = 2 * cid + p
            off = chunk * NP

            lax.fori_loop(0, B, zero_msg_a, 0)
            for i in range(RT // B):
                pltpu.sync_copy(msg_a, acc_s.at[pl.ds(row0 + i * B, B)])

            @pl.when(with_deg)
            def _():
                for i in range(RT // B):
                    pltpu.sync_copy(msg_a.at[0],
                                    deg_s.at[pl.ds(row0 + i * B, B)])

            plsc.subcore_barrier()

            def super_body(sb, _):
                pltpu.sync_copy(src_hbm.at[tid, pl.ds(sb * SBN, SBN)], src_v)
                pltpu.sync_copy(dst_hbm.at[tid, pl.ds(sb * SBN, SBN)], dst_v)
                pltpu.sync_copy(w_hbm.at[tid, pl.ds(sb * SBN, SBN)], w_v)

                def add_off(r, _):
                    for q in range(B // 16):
                        sl = pl.ds(q * 16, 16)
                        src_v[r, sl] = src_v[r, sl] + off
                    return 0

                lax.fori_loop(0, SBN, add_off, 0)

                gd = [None] * SBN
                sd = [None] * SBN
                gd[0] = pltpu.async_copy(xh_hbm.at[src_v.at[0]], msgs[0],
                                         gsems[0])
                for j in range(SBN):
                    cur = j % 2
                    buf = msgs[cur]
                    gd[j].wait()
                    if j + 1 < SBN:
                        nxt = (j + 1) % 2
                        if j >= 1:
                            sd[j - 1].wait()
                        gd[j + 1] = pltpu.async_copy(
                            xh_hbm.at[src_v.at[j + 1]], msgs[nxt],
                            gsems[nxt])
                    scale(buf, j)
                    sd[j] = pltpu.async_copy(buf, acc_s.at[dst_v.at[j]],
                                             ssems[cur], add=True)

                    @pl.when(with_deg)
                    def _(jj=j):
                        pltpu.async_copy(ones_v, deg_s.at[dst_v.at[jj]],
                                         sem_d, add=True)

                sd[SBN - 2].wait()
                sd[SBN - 1].wait()

                @pl.when(with_deg)
                def _():
                    for jj in range(SBN):
                        pltpu.make_async_copy(
                            ones_v, deg_s.at[dst_v.at[jj]], sem_d).wait()
                return 0

            lax.fori_loop(0, NSB, super_body, 0)
            plsc.subcore_barrier()

            for i in range(RT // B):
                pltpu.sync_copy(acc_s.at[pl.ds(row0 + i * B, B)], msg_a)
                pltpu.sync_copy(msg_a,
                                agg_out.at[chunk, pl.ds(row0 + i * B, B)])

            @pl.when(with_deg)
            def _():
                for i in range(RT // B):
                    pltpu.sync_copy(deg_s.at[pl.ds(row0 + i * B, B)],
                                    ones_v)
                    pltpu.sync_copy(ones_v,
                                    deg_out.at[pl.ds(row0 + i * B, B)])
            return 0

        lax.fori_loop(0, 2, do_pass, 0)

    f = pl.kernel(
        body,
        out_type=(jax.ShapeDtypeStruct((4, NP, DC), jnp.float32),
                  jax.ShapeDtypeStruct((NP,), jnp.float32)),
        mesh=mesh,
        scratch_types=[
            pltpu.VMEM((SBN, B), jnp.int32),
            pltpu.VMEM((SBN, B), jnp.int32),
            pltpu.VMEM((SBN, B), jnp.float32),
            pltpu.VMEM((B, DC), jnp.float32),
            pltpu.VMEM((B, DC), jnp.float32),
            pltpu.VMEM((B,), jnp.float32),
            pltpu.SemaphoreType.DMA,
            pltpu.SemaphoreType.DMA,
            pltpu.SemaphoreType.DMA,
            pltpu.SemaphoreType.DMA,
            pltpu.SemaphoreType.DMA,
            pltpu.VMEM_SHARED((NP, DC), jnp.float32),
            pltpu.VMEM_SHARED((NP,), jnp.float32),
        ],
    )
    return f(xh, src3, dst3, w3, ones1)


def _tc_body(a0, a1, a2, a3, deg1, x, h, win, wis, whn, whs, bi, bh, out):
    deg = jnp.maximum(deg1[...], 1.0)
    m0 = a0[...] / deg
    m1 = a1[...] / deg
    m2 = a2[...] / deg
    m3 = a3[...] / deg
    hp = lax.Precision.HIGHEST
    acc = jnp.dot(x[...], wis[...], precision=hp)
    acc += jnp.dot(h[...], whs[...], precision=hp)
    acc += jnp.dot(m0, win[0:DC, :], precision=hp)
    acc += jnp.dot(m1, win[DC:D, :], precision=hp)
    acc += jnp.dot(m2, whn[0:DC, :], precision=hp)
    acc += jnp.dot(m3, whn[DC:D, :], precision=hp)
    out[...] = jnp.tanh(acc + bi[...] + bh[...])


def _tc_dense(a0, a1, a2, a3, deg1, X, H, Wi_neigh, Wi_self, Wh_neigh,
              Wh_self, bi, bh):
    R = 400
    grid = (N // R,)

    def row_spec(w):
        return pl.BlockSpec((R, w), lambda i: (i, 0))

    def full_spec(r, c):
        return pl.BlockSpec((r, c), lambda i: (0, 0))

    return pl.pallas_call(
        _tc_body,
        grid=grid,
        in_specs=[
            row_spec(DC), row_spec(DC), row_spec(DC), row_spec(DC),
            pl.BlockSpec((R, 1), lambda i: (i, 0)),
            row_spec(D), row_spec(D),
            full_spec(D, D), full_spec(D, D), full_spec(D, D),
            full_spec(D, D), full_spec(1, D), full_spec(1, D),
        ],
        out_specs=row_spec(D),
        out_shape=jax.ShapeDtypeStruct((N, D), jnp.float32),
    )(a0, a1, a2, a3, deg1, X, H, Wi_neigh, Wi_self, Wh_neigh, Wh_self,
      bi.reshape(1, D), bh.reshape(1, D))


def kernel(X, edge_index, edge_weight, H, Wi_neigh, Wi_self, bi, Wh_neigh,
           Wh_self, bh):
    pad = ETP - ET
    src = jnp.pad(edge_index[0].reshape(NTILES, ET), ((0, 0), (0, pad)),
                  constant_values=0).reshape(NTILES, NB, B)
    dst = jnp.pad(edge_index[1].reshape(NTILES, ET), ((0, 0), (0, pad)),
                  constant_values=NP - 1).reshape(NTILES, NB, B)
    w = jnp.pad(edge_weight.reshape(NTILES, ET), ((0, 0), (0, pad)),
                constant_values=0.0).reshape(NTILES, NB, B)
    Xp = jnp.pad(X, ((0, NP - N), (0, 0)))
    Hp = jnp.pad(H, ((0, NP - N), (0, 0)))
    xh = jnp.concatenate([Xp[:, :DC], Xp[:, DC:], Hp[:, :DC], Hp[:, DC:]],
                         axis=0)
    ones1 = jnp.ones((B,), jnp.float32)

    agg, deg = _sc_segment_kernel(xh, src, dst, w, ones1)
    agg = agg[:, :N]
    deg1 = deg[:N].reshape(N, 1)
    return _tc_dense(agg[0], agg[1], agg[2], agg[3], deg1, X, H, Wi_neigh,
                     Wi_self, Wh_neigh, Wh_self, bi, bh)
